# baseline (speedup 1.0000x reference)
import functools

import jax
import jax.numpy as jnp
from jax import lax
from jax.experimental import pallas as pl
from jax.experimental.pallas import tpu as pltpu
from jax.experimental.pallas import tpu_sc as plsc

N_NODES = 10000
N_EDGES = 160000
D = 128
R = 5
T_PRED = 6
BLK = 512
NPAD = 10240
GRID = NPAD // BLK
TROWS = R * NPAD
CHUNK = 16
NCHUNK = 8
NS = 16
NC = 2
EPT = N_EDGES // NS
AB = 125
AB2 = 1250
ANB = EPT // AB2
CPT = N_EDGES // (NS * NC)
CNB = CPT // AB
SROWS = TROWS // NS
ZR = 800

_PREC = lax.Precision.HIGHEST


def _dot(a, b):
    return jnp.dot(a, b, preferred_element_type=jnp.float32, precision=_PREC)


def _pre_body(x_ref, w_ref, b_ref, m_ref, s_ref, root_ref, tbl_ref):
    x = (x_ref[:] - m_ref[:]) * s_ref[:]
    y = _dot(x, w_ref[:])
    root_ref[:] = y[:, :D] + b_ref[:]
    for c in range(NCHUNK):
        for r in range(R):
            lo = D + r * D + c * CHUNK
            tbl_ref[c, r] = y[:, lo:lo + CHUNK]


@functools.partial(jax.jit, static_argnums=())
def k_pre(x, wcat, bias, m, s):
    return pl.pallas_call(
        _pre_body,
        grid=(GRID,),
        in_specs=[
            pl.BlockSpec((BLK, D), lambda i: (i, 0)),
            pl.BlockSpec((D, D * (R + 1)), lambda i: (0, 0)),
            pl.BlockSpec((1, D), lambda i: (0, 0)),
            pl.BlockSpec((1, D), lambda i: (0, 0)),
            pl.BlockSpec((1, D), lambda i: (0, 0)),
        ],
        out_specs=[
            pl.BlockSpec((BLK, D), lambda i: (i, 0)),
            pl.BlockSpec((NCHUNK, R, BLK, CHUNK), lambda i: (0, 0, i, 0)),
        ],
        out_shape=[
            jax.ShapeDtypeStruct((NPAD, D), jnp.float32),
            jax.ShapeDtypeStruct((NCHUNK, R, NPAD, CHUNK), jnp.float32),
        ],
    )(x, wcat, bias, m, s)


def _comb_body(root_ref, acc_ref, cnt_ref, hpre_ref, stats_ref):
    i = pl.program_id(0)
    inv = 1.0 / jnp.maximum(cnt_ref[:], 1.0)
    h = root_ref[:]
    for r in range(R):
        m = jnp.concatenate([acc_ref[c, r] for c in range(NCHUNK)], axis=1)
        h = h + m * inv[r][:, None]
    rows = lax.broadcasted_iota(jnp.int32, (BLK, 1), 0) + i * BLK
    h = jnp.where(rows < N_NODES, h, 0.0)
    hpre_ref[:] = h

    @pl.when(i == 0)
    def _():
        stats_ref[:] = jnp.zeros_like(stats_ref)

    stats_ref[0:1] += jnp.sum(h, axis=0, keepdims=True)
    stats_ref[1:2] += jnp.sum(h * h, axis=0, keepdims=True)


@jax.jit
def k_comb(root, acc, cnt):
    return pl.pallas_call(
        _comb_body,
        grid=(GRID,),
        in_specs=[
            pl.BlockSpec((BLK, D), lambda i: (i, 0)),
            pl.BlockSpec((NCHUNK, R, BLK, CHUNK), lambda i: (0, 0, i, 0)),
            pl.BlockSpec((R, BLK), lambda i: (0, i)),
        ],
        out_specs=[
            pl.BlockSpec((BLK, D), lambda i: (i, 0)),
            pl.BlockSpec((2, D), lambda i: (0, 0)),
        ],
        out_shape=[
            jax.ShapeDtypeStruct((NPAD, D), jnp.float32),
            jax.ShapeDtypeStruct((2, D), jnp.float32),
        ],
    )(root, acc, cnt)


def _bn_body(h_ref, mu_ref, rs_ref, g_ref, b_ref, a_ref, res_ref, o_ref):
    hb = (h_ref[:] - mu_ref[:]) * rs_ref[:] * g_ref[:] + b_ref[:]
    o = jnp.where(hb >= 0, hb, a_ref[0, 0] * hb)
    o_ref[:] = o + res_ref[:]


def _bn0_body(h_ref, mu_ref, rs_ref, g_ref, b_ref, a_ref, o_ref):
    hb = (h_ref[:] - mu_ref[:]) * rs_ref[:] * g_ref[:] + b_ref[:]
    o_ref[:] = jnp.where(hb >= 0, hb, a_ref[0, 0] * hb)


def _bn_specs(nin):
    vec = pl.BlockSpec((1, D), lambda i: (0, 0))
    blk = pl.BlockSpec((BLK, D), lambda i: (i, 0))
    scal = pl.BlockSpec((1, 1), lambda i: (0, 0))
    specs = [blk, vec, vec, vec, vec, scal] + [blk] * (nin - 6)
    return specs


@jax.jit
def k_bn(h, mu, rs, g, b, a, res):
    return pl.pallas_call(
        _bn_body,
        grid=(GRID,),
        in_specs=_bn_specs(7),
        out_specs=pl.BlockSpec((BLK, D), lambda i: (i, 0)),
        out_shape=jax.ShapeDtypeStruct((NPAD, D), jnp.float32),
    )(h, mu, rs, g, b, a, res)


@jax.jit
def k_bn0(h, mu, rs, g, b, a):
    return pl.pallas_call(
        _bn0_body,
        grid=(GRID,),
        in_specs=_bn_specs(6),
        out_specs=pl.BlockSpec((BLK, D), lambda i: (i, 0)),
        out_shape=jax.ShapeDtypeStruct((NPAD, D), jnp.float32),
    )(h, mu, rs, g, b, a)


def _mm_body(x_ref, w_ref, b_ref, o_ref):
    o_ref[:] = _dot(x_ref[:], w_ref[:]) + b_ref[:]


@jax.jit
def k_mm(x, w, b):
    ko, no = w.shape
    return pl.pallas_call(
        _mm_body,
        grid=(GRID,),
        in_specs=[
            pl.BlockSpec((BLK, ko), lambda i: (i, 0)),
            pl.BlockSpec((ko, no), lambda i: (0, 0)),
            pl.BlockSpec((1, no), lambda i: (0, 0)),
        ],
        out_specs=pl.BlockSpec((BLK, no), lambda i: (i, 0)),
        out_shape=jax.ShapeDtypeStruct((NPAD, no), jnp.float32),
    )(x, w, b)


def _mlp_body(x_ref, w1_ref, b1_ref, g_ref, bb_ref, w2_ref, b2_ref, o_ref):
    h = jnp.maximum(_dot(x_ref[:], w1_ref[:]) + b1_ref[:], 0.0)
    mu = jnp.mean(h, axis=1, keepdims=True)
    var = jnp.mean((h - mu) ** 2, axis=1, keepdims=True)
    hn = g_ref[:] * (h - mu) / jnp.sqrt(var + 1e-5) + bb_ref[:]
    o_ref[:] = _dot(hn, w2_ref[:]) + b2_ref[:]


@jax.jit
def k_mlp(x, w1, b1, g, bb, w2, b2):
    ki, kh = w1.shape
    ko = w2.shape[1]
    return pl.pallas_call(
        _mlp_body,
        grid=(GRID,),
        in_specs=[
            pl.BlockSpec((BLK, ki), lambda i: (i, 0)),
            pl.BlockSpec((ki, kh), lambda i: (0, 0)),
            pl.BlockSpec((1, kh), lambda i: (0, 0)),
            pl.BlockSpec((1, kh), lambda i: (0, 0)),
            pl.BlockSpec((1, kh), lambda i: (0, 0)),
            pl.BlockSpec((kh, ko), lambda i: (0, 0)),
            pl.BlockSpec((1, ko), lambda i: (0, 0)),
        ],
        out_specs=pl.BlockSpec((BLK, ko), lambda i: (i, 0)),
        out_shape=jax.ShapeDtypeStruct((NPAD, ko), jnp.float32),
    )(x, w1, b1, g, bb, w2, b2)


def _lstm_body(xs_ref, h0_ref, wih_ref, whh_ref, bs_ref,
               hw1_ref, hb1_ref, hw2_ref, hb2_ref, o_ref):
    h = h0_ref[:]
    c = jnp.zeros((BLK, D), jnp.float32)
    ps = []
    for t in range(T_PRED):
        gates = _dot(xs_ref[t], wih_ref[:]) + _dot(h, whh_ref[:]) + bs_ref[:]
        ii = gates[:, 0:D]
        ff = gates[:, D:2 * D]
        gg = gates[:, 2 * D:3 * D]
        oo = gates[:, 3 * D:4 * D]
        c = jax.nn.sigmoid(ff) * c + jax.nn.sigmoid(ii) * jnp.tanh(gg)
        h = jax.nn.sigmoid(oo) * jnp.tanh(c)
        hd = jnp.maximum(_dot(h, hw1_ref[t]) + hb1_ref[t], 0.0)
        p = jnp.sum(hd * hw2_ref[t], axis=1, keepdims=True) + hb2_ref[t]
        ps.append(p)
    o_ref[:] = jnp.concatenate(ps, axis=1)


@jax.jit
def k_lstm(xs, h0, wih, whh, bs, hw1, hb1, hw2, hb2):
    hh = 64
    return pl.pallas_call(
        _lstm_body,
        grid=(GRID,),
        in_specs=[
            pl.BlockSpec((T_PRED, BLK, D), lambda i: (0, i, 0)),
            pl.BlockSpec((BLK, D), lambda i: (i, 0)),
            pl.BlockSpec((D, 4 * D), lambda i: (0, 0)),
            pl.BlockSpec((D, 4 * D), lambda i: (0, 0)),
            pl.BlockSpec((1, 4 * D), lambda i: (0, 0)),
            pl.BlockSpec((T_PRED, D, hh), lambda i: (0, 0, 0)),
            pl.BlockSpec((T_PRED, 1, hh), lambda i: (0, 0, 0)),
            pl.BlockSpec((T_PRED, 1, hh), lambda i: (0, 0, 0)),
            pl.BlockSpec((T_PRED, 1, 1), lambda i: (0, 0, 0)),
        ],
        out_specs=pl.BlockSpec((BLK, T_PRED), lambda i: (i, 0)),
        out_shape=jax.ShapeDtypeStruct((NPAD, T_PRED), jnp.float32),
    )(xs, h0, wih, whh, bs, hw1, hb1, hw2, hb2)


def _agg_body(tbl_ref, g3_ref, h3_ref, out_ref, gi, hi, rows0, rows1, zb,
              acc, sem0, sem1):
    c = lax.axis_index("c")
    s = lax.axis_index("s")
    pltpu.sync_copy(g3_ref.at[s], gi)
    pltpu.sync_copy(h3_ref.at[s], hi)

    def zero_body(i, carry):
        zb[i, pl.ds(0, 16)] = jnp.zeros((16,), jnp.float32)
        return carry

    lax.fori_loop(0, ZR, zero_body, 0)
    for p in range(NCHUNK // NC):
        q = p * NC + c

        def zcp(j, carry):
            pltpu.sync_copy(zb, acc.at[pl.ds(s * SROWS + j * ZR, ZR)])
            return carry

        lax.fori_loop(0, SROWS // ZR, zcp, 0)
        plsc.subcore_barrier()
        tq = tbl_ref.at[q]
        pltpu.async_copy(tq.at[gi.at[0]], rows0, sem0)

        def body2(jj, carry):
            j0 = jj * 2
            pltpu.async_copy(tq.at[gi.at[j0 + 1]], rows1, sem1)
            pltpu.make_async_copy(tq.at[gi.at[j0]], rows0, sem0).wait()
            pltpu.sync_copy(rows0, acc.at[hi.at[j0]], add=True)
            j2 = jnp.minimum(j0 + 2, ANB - 1)
            pltpu.async_copy(tq.at[gi.at[j2]], rows0, sem0)
            pltpu.make_async_copy(tq.at[gi.at[j0 + 1]], rows1, sem1).wait()
            pltpu.sync_copy(rows1, acc.at[hi.at[j0 + 1]], add=True)
            return carry

        lax.fori_loop(0, ANB // 2, body2, 0)
        pltpu.make_async_copy(tq.at[gi.at[ANB - 1]], rows0, sem0).wait()
        plsc.subcore_barrier()
        pltpu.sync_copy(acc.at[pl.ds(s * SROWS, SROWS)],
                        out_ref.at[q].at[pl.ds(s * SROWS, SROWS)])
        plsc.subcore_barrier()


@functools.lru_cache(maxsize=None)
def _sc_agg_kernel():
    mesh = plsc.VectorSubcoreMesh(core_axis_name="c", subcore_axis_name="s")
    return pl.kernel(
        _agg_body,
        out_type=jax.ShapeDtypeStruct((NCHUNK, TROWS, CHUNK), jnp.float32),
        mesh=mesh,
        compiler_params=pltpu.CompilerParams(use_tc_tiling_on_sc=False),
        scratch_types=[
            pltpu.VMEM((ANB, AB2), jnp.int32),
            pltpu.VMEM((ANB, AB2), jnp.int32),
            pltpu.VMEM((AB2, CHUNK), jnp.float32),
            pltpu.VMEM((AB2, CHUNK), jnp.float32),
            pltpu.VMEM((ZR, CHUNK), jnp.float32),
            pltpu.VMEM_SHARED((TROWS, CHUNK), jnp.float32),
            pltpu.SemaphoreType.DMA,
            pltpu.SemaphoreType.DMA,
        ],
    )


def sc_agg(tbl3, gidx, h3):
    return _sc_agg_kernel()(tbl3, gidx, h3)


def _cnt_body(h32_ref, ones_ref, out_ref, hi, ones_v, zb, cacc, sem):
    c = lax.axis_index("c")
    s = lax.axis_index("s")
    w = c * NS + s
    pltpu.sync_copy(h32_ref.at[w], hi)
    pltpu.sync_copy(ones_ref, ones_v)

    def zero_body(i, carry):
        zb[i, pl.ds(0, 16)] = jnp.zeros((16,), jnp.float32)
        return carry

    lax.fori_loop(0, ZR, zero_body, 0)

    def zcp(j, carry):
        pltpu.sync_copy(zb, cacc.at[pl.ds(s * SROWS + j * ZR, ZR)])
        return carry

    lax.fori_loop(0, SROWS // ZR, zcp, 0)
    plsc.subcore_barrier()

    def body(j, carry):
        pltpu.sync_copy(ones_v.at[pl.ds(0, AB)], cacc.at[hi.at[j]], add=True)
        return carry

    lax.fori_loop(0, CNB, body, 0)
    plsc.subcore_barrier()
    pltpu.sync_copy(cacc.at[pl.ds(s * SROWS, SROWS)],
                    out_ref.at[c].at[pl.ds(s * SROWS, SROWS)])


@functools.lru_cache(maxsize=None)
def _sc_cnt_kernel():
    mesh = plsc.VectorSubcoreMesh(core_axis_name="c", subcore_axis_name="s")
    return pl.kernel(
        _cnt_body,
        out_type=jax.ShapeDtypeStruct((NC, TROWS, 16), jnp.float32),
        mesh=mesh,
        compiler_params=pltpu.CompilerParams(use_tc_tiling_on_sc=False),
        scratch_types=[
            pltpu.VMEM((CNB, AB), jnp.int32),
            pltpu.VMEM((128, 16), jnp.float32),
            pltpu.VMEM((ZR, 16), jnp.float32),
            pltpu.VMEM_SHARED((TROWS, 16), jnp.float32),
            pltpu.SemaphoreType.DMA,
        ],
    )


def sc_cnt(h32, ones):
    return _sc_cnt_kernel()(h32, ones)


def _layer_norm(h, g, b):
    mu = h.mean(-1, keepdims=True)
    var = ((h - mu) ** 2).mean(-1, keepdims=True)
    return g * (h - mu) / jnp.sqrt(var + 1e-5) + b


def _mlp_small(x, p):
    h = jnp.maximum(x @ p['w1'] + p['b1'], 0.0)
    h = _layer_norm(h, p['ln_g'], p['ln_b'])
    return h @ p['w2'] + p['b2']


def _row(v):
    return v.reshape(1, -1)


def kernel(x_seq, edge_index, edge_attr, graph_global_env_features,
           timeline_time_features, params):
    f32 = jnp.float32
    src = edge_index[0].astype(jnp.int32)
    dst = edge_index[1].astype(jnp.int32)
    etype = edge_attr[:, 4].astype(jnp.int32)
    gidx = (etype * NPAD + src).reshape(NS, ANB, AB2)
    hflat = etype * NPAD + dst
    h3 = hflat.reshape(NS, ANB, AB2)
    h32 = hflat.reshape(NS * NC, CNB, AB)

    cnt2 = sc_cnt(h32, jnp.ones((128, 16), f32))
    cnt = (cnt2[0, :, 0] + cnt2[1, :, 0]).reshape(R, NPAD)

    xpad = jnp.pad(x_seq, ((0, 0), (0, NPAD - N_NODES), (0, 0)))
    mean = _row(params['feat_mean'])
    std = _row(params['feat_std'] + 1e-8)
    inv_std = 1.0 / std
    zero_m = jnp.zeros((1, D), f32)
    one_s = jnp.ones((1, D), f32)

    def wcat_of(bp):
        return jnp.concatenate([bp['w_root']] +
                               [bp['w_rel'][r] for r in range(R)], axis=1)

    def conv(x, bp, m, s):
        root, tbl = k_pre(x, wcat_of(bp), _row(bp['bias']), m, s)
        acc = sc_agg(tbl.reshape(NCHUNK, TROWS, CHUNK), gidx, h3)
        h_pre, stats = k_comb(root, acc.reshape(NCHUNK, R, NPAD, CHUNK), cnt)
        mu = stats[0] / N_NODES
        var = stats[1] / N_NODES - mu * mu
        rstd = 1.0 / jnp.sqrt(var + 1e-5)
        return h_pre, _row(mu), _row(rstd)

    def gen(x, gp):
        b0 = gp['block0']
        hp, mu, rstd = conv(x, b0, mean, inv_std)
        a0 = b0['prelu'].reshape(1, 1)
        h0 = k_bn0(hp, mu, rstd, _row(b0['bn_g']), _row(b0['bn_b']), a0)
        b1 = gp['block1']
        hp1, mu1, rstd1 = conv(h0, b1, zero_m, one_s)
        a1 = b1['prelu'].reshape(1, 1)
        h1 = k_bn(hp1, mu1, rstd1, _row(b1['bn_g']), _row(b1['bn_b']), a1, h0)
        return k_mm(h1, gp['fc_w'], _row(gp['fc_b']))

    f0 = gen(xpad[0], params['gen_h0'])
    pe = params['h0_enc']
    h0 = k_mlp(f0, pe['w1'], _row(pe['b1']), _row(pe['ln_g']),
               _row(pe['ln_b']), pe['w2'], _row(pe['b2']))

    env_emb = _mlp_small(graph_global_env_features, params['env_enc'])
    tim_emb = _mlp_small(timeline_time_features, params['time_enc'])
    pf = params['fusion']
    w1_ft = pf['w1'][:D]
    b1_eff = (_row(pf['b1']) + env_emb @ pf['w1'][D:D + 16]
              + tim_emb @ pf['w1'][D + 16:])

    fused = []
    for t in range(T_PRED):
        ft = gen(xpad[t + 1], params['gen_seq'])
        fused.append(k_mlp(ft, w1_ft, b1_eff[t:t + 1], _row(pf['ln_g']),
                           _row(pf['ln_b']), pf['w2'], _row(pf['b2'])))
    xs = jnp.stack(fused, 0)

    lp = params['lstm']
    bs = _row(lp['b_ih'] + lp['b_hh'])
    hw1 = jnp.stack([params['heads'][t]['w1'] for t in range(T_PRED)], 0)
    hb1 = jnp.stack([params['heads'][t]['b1'].reshape(1, -1)
                     for t in range(T_PRED)], 0)
    hw2 = jnp.stack([params['heads'][t]['w2'].reshape(1, -1)
                     for t in range(T_PRED)], 0)
    hb2 = jnp.stack([params['heads'][t]['b2'].reshape(1, 1)
                     for t in range(T_PRED)], 0)
    preds = k_lstm(xs, h0, lp['w_ih'].T, lp['w_hh'].T, bs,
                   hw1, hb1, hw2, hb2)
    return preds[:N_NODES]


# speedup vs baseline: 1.6044x; 1.6044x over previous
import functools

import jax
import jax.numpy as jnp
from jax import lax
from jax.experimental import pallas as pl
from jax.experimental.pallas import tpu as pltpu
from jax.experimental.pallas import tpu_sc as plsc

N_NODES = 10000
N_EDGES = 160000
D = 128
R = 5
T_PRED = 6
BLK = 512
NPAD = 10240
GRID = NPAD // BLK
TROWS = R * NPAD
CHUNK = 32
NCHUNK = 4
NS = 16
NC = 2
EPT = N_EDGES // NS
AB = 125
WNB = 40
NW = EPT // (WNB * AB)
CPT = N_EDGES // (NS * NC)
CNB = CPT // AB
SROWS = TROWS // NS
ZR = 100

_PREC = lax.Precision.HIGHEST


def _dot(a, b):
    return jnp.dot(a, b, preferred_element_type=jnp.float32, precision=_PREC)


def _pre_body(x_ref, w_ref, b_ref, m_ref, s_ref, root_ref, tbl_ref):
    x = (x_ref[:] - m_ref[:]) * s_ref[:]
    y = _dot(x, w_ref[:])
    root_ref[:] = y[:, :D] + b_ref[:]
    for c in range(NCHUNK):
        for r in range(R):
            lo = D + r * D + c * CHUNK
            tbl_ref[c, r] = y[:, lo:lo + CHUNK]


@functools.partial(jax.jit, static_argnums=())
def k_pre(x, wcat, bias, m, s):
    return pl.pallas_call(
        _pre_body,
        grid=(GRID,),
        in_specs=[
            pl.BlockSpec((BLK, D), lambda i: (i, 0)),
            pl.BlockSpec((D, D * (R + 1)), lambda i: (0, 0)),
            pl.BlockSpec((1, D), lambda i: (0, 0)),
            pl.BlockSpec((1, D), lambda i: (0, 0)),
            pl.BlockSpec((1, D), lambda i: (0, 0)),
        ],
        out_specs=[
            pl.BlockSpec((BLK, D), lambda i: (i, 0)),
            pl.BlockSpec((NCHUNK, R, BLK, CHUNK), lambda i: (0, 0, i, 0)),
        ],
        out_shape=[
            jax.ShapeDtypeStruct((NPAD, D), jnp.float32),
            jax.ShapeDtypeStruct((NCHUNK, R, NPAD, CHUNK), jnp.float32),
        ],
    )(x, wcat, bias, m, s)


def _comb_body(root_ref, acc_ref, cnt_ref, hpre_ref, stats_ref):
    i = pl.program_id(0)
    inv = 1.0 / jnp.maximum(cnt_ref[:], 1.0)
    h = root_ref[:]
    for r in range(R):
        m = jnp.concatenate([acc_ref[c, r] for c in range(NCHUNK)], axis=1)
        h = h + m * inv[r][:, None]
    rows = lax.broadcasted_iota(jnp.int32, (BLK, 1), 0) + i * BLK
    h = jnp.where(rows < N_NODES, h, 0.0)
    hpre_ref[:] = h

    @pl.when(i == 0)
    def _():
        stats_ref[:] = jnp.zeros_like(stats_ref)

    stats_ref[0:1] += jnp.sum(h, axis=0, keepdims=True)
    stats_ref[1:2] += jnp.sum(h * h, axis=0, keepdims=True)


@jax.jit
def k_comb(root, acc, cnt):
    return pl.pallas_call(
        _comb_body,
        grid=(GRID,),
        in_specs=[
            pl.BlockSpec((BLK, D), lambda i: (i, 0)),
            pl.BlockSpec((NCHUNK, R, BLK, CHUNK), lambda i: (0, 0, i, 0)),
            pl.BlockSpec((R, BLK), lambda i: (0, i)),
        ],
        out_specs=[
            pl.BlockSpec((BLK, D), lambda i: (i, 0)),
            pl.BlockSpec((2, D), lambda i: (0, 0)),
        ],
        out_shape=[
            jax.ShapeDtypeStruct((NPAD, D), jnp.float32),
            jax.ShapeDtypeStruct((2, D), jnp.float32),
        ],
    )(root, acc, cnt)


def _bn_body(h_ref, mu_ref, rs_ref, g_ref, b_ref, a_ref, res_ref, o_ref):
    hb = (h_ref[:] - mu_ref[:]) * rs_ref[:] * g_ref[:] + b_ref[:]
    o = jnp.where(hb >= 0, hb, a_ref[0, 0] * hb)
    o_ref[:] = o + res_ref[:]


def _bn0_body(h_ref, mu_ref, rs_ref, g_ref, b_ref, a_ref, o_ref):
    hb = (h_ref[:] - mu_ref[:]) * rs_ref[:] * g_ref[:] + b_ref[:]
    o_ref[:] = jnp.where(hb >= 0, hb, a_ref[0, 0] * hb)


def _bn_specs(nin):
    vec = pl.BlockSpec((1, D), lambda i: (0, 0))
    blk = pl.BlockSpec((BLK, D), lambda i: (i, 0))
    scal = pl.BlockSpec((1, 1), lambda i: (0, 0))
    specs = [blk, vec, vec, vec, vec, scal] + [blk] * (nin - 6)
    return specs


@jax.jit
def k_bn(h, mu, rs, g, b, a, res):
    return pl.pallas_call(
        _bn_body,
        grid=(GRID,),
        in_specs=_bn_specs(7),
        out_specs=pl.BlockSpec((BLK, D), lambda i: (i, 0)),
        out_shape=jax.ShapeDtypeStruct((NPAD, D), jnp.float32),
    )(h, mu, rs, g, b, a, res)


@jax.jit
def k_bn0(h, mu, rs, g, b, a):
    return pl.pallas_call(
        _bn0_body,
        grid=(GRID,),
        in_specs=_bn_specs(6),
        out_specs=pl.BlockSpec((BLK, D), lambda i: (i, 0)),
        out_shape=jax.ShapeDtypeStruct((NPAD, D), jnp.float32),
    )(h, mu, rs, g, b, a)


def _mm_body(x_ref, w_ref, b_ref, o_ref):
    o_ref[:] = _dot(x_ref[:], w_ref[:]) + b_ref[:]


@jax.jit
def k_mm(x, w, b):
    ko, no = w.shape
    return pl.pallas_call(
        _mm_body,
        grid=(GRID,),
        in_specs=[
            pl.BlockSpec((BLK, ko), lambda i: (i, 0)),
            pl.BlockSpec((ko, no), lambda i: (0, 0)),
            pl.BlockSpec((1, no), lambda i: (0, 0)),
        ],
        out_specs=pl.BlockSpec((BLK, no), lambda i: (i, 0)),
        out_shape=jax.ShapeDtypeStruct((NPAD, no), jnp.float32),
    )(x, w, b)


def _mlp_body(x_ref, w1_ref, b1_ref, g_ref, bb_ref, w2_ref, b2_ref, o_ref):
    h = jnp.maximum(_dot(x_ref[:], w1_ref[:]) + b1_ref[:], 0.0)
    mu = jnp.mean(h, axis=1, keepdims=True)
    var = jnp.mean((h - mu) ** 2, axis=1, keepdims=True)
    hn = g_ref[:] * (h - mu) / jnp.sqrt(var + 1e-5) + bb_ref[:]
    o_ref[:] = _dot(hn, w2_ref[:]) + b2_ref[:]


@jax.jit
def k_mlp(x, w1, b1, g, bb, w2, b2):
    ki, kh = w1.shape
    ko = w2.shape[1]
    return pl.pallas_call(
        _mlp_body,
        grid=(GRID,),
        in_specs=[
            pl.BlockSpec((BLK, ki), lambda i: (i, 0)),
            pl.BlockSpec((ki, kh), lambda i: (0, 0)),
            pl.BlockSpec((1, kh), lambda i: (0, 0)),
            pl.BlockSpec((1, kh), lambda i: (0, 0)),
            pl.BlockSpec((1, kh), lambda i: (0, 0)),
            pl.BlockSpec((kh, ko), lambda i: (0, 0)),
            pl.BlockSpec((1, ko), lambda i: (0, 0)),
        ],
        out_specs=pl.BlockSpec((BLK, ko), lambda i: (i, 0)),
        out_shape=jax.ShapeDtypeStruct((NPAD, ko), jnp.float32),
    )(x, w1, b1, g, bb, w2, b2)


def _lstm_body(xs_ref, h0_ref, wih_ref, whh_ref, bs_ref,
               hw1_ref, hb1_ref, hw2_ref, hb2_ref, o_ref):
    h = h0_ref[:]
    c = jnp.zeros((BLK, D), jnp.float32)
    ps = []
    for t in range(T_PRED):
        gates = _dot(xs_ref[t], wih_ref[:]) + _dot(h, whh_ref[:]) + bs_ref[:]
        ii = gates[:, 0:D]
        ff = gates[:, D:2 * D]
        gg = gates[:, 2 * D:3 * D]
        oo = gates[:, 3 * D:4 * D]
        c = jax.nn.sigmoid(ff) * c + jax.nn.sigmoid(ii) * jnp.tanh(gg)
        h = jax.nn.sigmoid(oo) * jnp.tanh(c)
        hd = jnp.maximum(_dot(h, hw1_ref[t]) + hb1_ref[t], 0.0)
        p = jnp.sum(hd * hw2_ref[t], axis=1, keepdims=True) + hb2_ref[t]
        ps.append(p)
    o_ref[:] = jnp.concatenate(ps, axis=1)


@jax.jit
def k_lstm(xs, h0, wih, whh, bs, hw1, hb1, hw2, hb2):
    hh = 64
    return pl.pallas_call(
        _lstm_body,
        grid=(GRID,),
        in_specs=[
            pl.BlockSpec((T_PRED, BLK, D), lambda i: (0, i, 0)),
            pl.BlockSpec((BLK, D), lambda i: (i, 0)),
            pl.BlockSpec((D, 4 * D), lambda i: (0, 0)),
            pl.BlockSpec((D, 4 * D), lambda i: (0, 0)),
            pl.BlockSpec((1, 4 * D), lambda i: (0, 0)),
            pl.BlockSpec((T_PRED, D, hh), lambda i: (0, 0, 0)),
            pl.BlockSpec((T_PRED, 1, hh), lambda i: (0, 0, 0)),
            pl.BlockSpec((T_PRED, 1, hh), lambda i: (0, 0, 0)),
            pl.BlockSpec((T_PRED, 1, 1), lambda i: (0, 0, 0)),
        ],
        out_specs=pl.BlockSpec((BLK, T_PRED), lambda i: (i, 0)),
        out_shape=jax.ShapeDtypeStruct((NPAD, T_PRED), jnp.float32),
    )(xs, h0, wih, whh, bs, hw1, hb1, hw2, hb2)


def _agg_body(tbl_ref, g3_ref, h3_ref, out_ref, gi, hi, rows0, rows1, zb,
              acc, sem0, sem1):
    c = lax.axis_index("c")
    s = lax.axis_index("s")

    def zero_body(i, carry):
        zb[i, pl.ds(0, 16)] = jnp.zeros((16,), jnp.float32)
        zb[i, pl.ds(16, 16)] = jnp.zeros((16,), jnp.float32)
        return carry

    lax.fori_loop(0, ZR, zero_body, 0)
    for p in range(NCHUNK // NC):
        q = p * NC + c

        def zfire(j, carry):
            pltpu.async_copy(zb, acc.at[pl.ds(s * SROWS + j * ZR, ZR)], sem0)
            return carry

        def zdrain(j, carry):
            pltpu.make_async_copy(zb, acc.at[pl.ds(s * SROWS, ZR)],
                                  sem0).wait()
            return carry

        lax.fori_loop(0, SROWS // ZR, zfire, 0)
        lax.fori_loop(0, SROWS // ZR, zdrain, 0)
        plsc.subcore_barrier()
        tq = tbl_ref.at[q]
        for w in range(NW):
            pltpu.sync_copy(g3_ref.at[s, w], gi)
            pltpu.sync_copy(h3_ref.at[s, w], hi)
            pltpu.async_copy(tq.at[gi.at[0]], rows0, sem0)

            def body2(jj, carry):
                j0 = jj * 2
                pltpu.async_copy(tq.at[gi.at[j0 + 1]], rows1, sem1)
                pltpu.make_async_copy(tq.at[gi.at[j0]], rows0, sem0).wait()
                pltpu.sync_copy(rows0, acc.at[hi.at[j0]], add=True)
                j2 = jnp.minimum(j0 + 2, WNB - 1)
                pltpu.async_copy(tq.at[gi.at[j2]], rows0, sem0)
                pltpu.make_async_copy(tq.at[gi.at[j0 + 1]], rows1,
                                      sem1).wait()
                pltpu.sync_copy(rows1, acc.at[hi.at[j0 + 1]], add=True)
                return carry

            lax.fori_loop(0, WNB // 2, body2, 0)
            pltpu.make_async_copy(tq.at[gi.at[WNB - 1]], rows0, sem0).wait()
        plsc.subcore_barrier()
        pltpu.sync_copy(acc.at[pl.ds(s * SROWS, SROWS)],
                        out_ref.at[q].at[pl.ds(s * SROWS, SROWS)])
        plsc.subcore_barrier()


@functools.lru_cache(maxsize=None)
def _sc_agg_kernel():
    mesh = plsc.VectorSubcoreMesh(core_axis_name="c", subcore_axis_name="s")
    return pl.kernel(
        _agg_body,
        out_type=jax.ShapeDtypeStruct((NCHUNK, TROWS, CHUNK), jnp.float32),
        mesh=mesh,
        compiler_params=pltpu.CompilerParams(use_tc_tiling_on_sc=False),
        scratch_types=[
            pltpu.VMEM((WNB, AB), jnp.int32),
            pltpu.VMEM((WNB, AB), jnp.int32),
            pltpu.VMEM((AB, CHUNK), jnp.float32),
            pltpu.VMEM((AB, CHUNK), jnp.float32),
            pltpu.VMEM((ZR, CHUNK), jnp.float32),
            pltpu.VMEM_SHARED((TROWS, CHUNK), jnp.float32),
            pltpu.SemaphoreType.DMA,
            pltpu.SemaphoreType.DMA,
        ],
    )


def sc_agg(tbl3, gidx, h3):
    return _sc_agg_kernel()(tbl3, gidx, h3)


def _cnt_body(h32_ref, ones_ref, out_ref, hi, ones_v, zb, cacc, sem):
    c = lax.axis_index("c")
    s = lax.axis_index("s")
    w = c * NS + s
    pltpu.sync_copy(h32_ref.at[w], hi)
    pltpu.sync_copy(ones_ref, ones_v)

    def zero_body(i, carry):
        zb[i, pl.ds(0, 16)] = jnp.zeros((16,), jnp.float32)
        return carry

    lax.fori_loop(0, ZR, zero_body, 0)

    def zcp(j, carry):
        pltpu.sync_copy(zb, cacc.at[pl.ds(s * SROWS + j * ZR, ZR)])
        return carry

    lax.fori_loop(0, SROWS // ZR, zcp, 0)
    plsc.subcore_barrier()

    def body(j, carry):
        pltpu.sync_copy(ones_v.at[pl.ds(0, AB)], cacc.at[hi.at[j]], add=True)
        return carry

    lax.fori_loop(0, CNB, body, 0)
    plsc.subcore_barrier()
    pltpu.sync_copy(cacc.at[pl.ds(s * SROWS, SROWS)],
                    out_ref.at[c].at[pl.ds(s * SROWS, SROWS)])


@functools.lru_cache(maxsize=None)
def _sc_cnt_kernel():
    mesh = plsc.VectorSubcoreMesh(core_axis_name="c", subcore_axis_name="s")
    return pl.kernel(
        _cnt_body,
        out_type=jax.ShapeDtypeStruct((NC, TROWS, 16), jnp.float32),
        mesh=mesh,
        compiler_params=pltpu.CompilerParams(use_tc_tiling_on_sc=False),
        scratch_types=[
            pltpu.VMEM((CNB, AB), jnp.int32),
            pltpu.VMEM((128, 16), jnp.float32),
            pltpu.VMEM((ZR, 16), jnp.float32),
            pltpu.VMEM_SHARED((TROWS, 16), jnp.float32),
            pltpu.SemaphoreType.DMA,
        ],
    )


def sc_cnt(h32, ones):
    return _sc_cnt_kernel()(h32, ones)


def _layer_norm(h, g, b):
    mu = h.mean(-1, keepdims=True)
    var = ((h - mu) ** 2).mean(-1, keepdims=True)
    return g * (h - mu) / jnp.sqrt(var + 1e-5) + b


def _mlp_small(x, p):
    h = jnp.maximum(x @ p['w1'] + p['b1'], 0.0)
    h = _layer_norm(h, p['ln_g'], p['ln_b'])
    return h @ p['w2'] + p['b2']


def _row(v):
    return v.reshape(1, -1)


def kernel(x_seq, edge_index, edge_attr, graph_global_env_features,
           timeline_time_features, params):
    f32 = jnp.float32
    src = edge_index[0].astype(jnp.int32)
    dst = edge_index[1].astype(jnp.int32)
    etype = edge_attr[:, 4].astype(jnp.int32)
    gidx = (etype * NPAD + src).reshape(NS, NW, WNB, AB)
    hflat = etype * NPAD + dst
    h3 = hflat.reshape(NS, NW, WNB, AB)
    h32 = hflat.reshape(NS * NC, CNB, AB)

    cnt2 = sc_cnt(h32, jnp.ones((128, 16), f32))
    cnt = (cnt2[0, :, 0] + cnt2[1, :, 0]).reshape(R, NPAD)

    xpad = jnp.pad(x_seq, ((0, 0), (0, NPAD - N_NODES), (0, 0)))
    mean = _row(params['feat_mean'])
    std = _row(params['feat_std'] + 1e-8)
    inv_std = 1.0 / std
    zero_m = jnp.zeros((1, D), f32)
    one_s = jnp.ones((1, D), f32)

    def wcat_of(bp):
        return jnp.concatenate([bp['w_root']] +
                               [bp['w_rel'][r] for r in range(R)], axis=1)

    def conv(x, bp, m, s):
        root, tbl = k_pre(x, wcat_of(bp), _row(bp['bias']), m, s)
        acc = sc_agg(tbl.reshape(NCHUNK, TROWS, CHUNK), gidx, h3)
        h_pre, stats = k_comb(root, acc.reshape(NCHUNK, R, NPAD, CHUNK), cnt)
        mu = stats[0] / N_NODES
        var = stats[1] / N_NODES - mu * mu
        rstd = 1.0 / jnp.sqrt(var + 1e-5)
        return h_pre, _row(mu), _row(rstd)

    def gen(x, gp):
        b0 = gp['block0']
        hp, mu, rstd = conv(x, b0, mean, inv_std)
        a0 = b0['prelu'].reshape(1, 1)
        h0 = k_bn0(hp, mu, rstd, _row(b0['bn_g']), _row(b0['bn_b']), a0)
        b1 = gp['block1']
        hp1, mu1, rstd1 = conv(h0, b1, zero_m, one_s)
        a1 = b1['prelu'].reshape(1, 1)
        h1 = k_bn(hp1, mu1, rstd1, _row(b1['bn_g']), _row(b1['bn_b']), a1, h0)
        return k_mm(h1, gp['fc_w'], _row(gp['fc_b']))

    f0 = gen(xpad[0], params['gen_h0'])
    pe = params['h0_enc']
    h0 = k_mlp(f0, pe['w1'], _row(pe['b1']), _row(pe['ln_g']),
               _row(pe['ln_b']), pe['w2'], _row(pe['b2']))

    env_emb = _mlp_small(graph_global_env_features, params['env_enc'])
    tim_emb = _mlp_small(timeline_time_features, params['time_enc'])
    pf = params['fusion']
    w1_ft = pf['w1'][:D]
    b1_eff = (_row(pf['b1']) + env_emb @ pf['w1'][D:D + 16]
              + tim_emb @ pf['w1'][D + 16:])

    fused = []
    for t in range(T_PRED):
        ft = gen(xpad[t + 1], params['gen_seq'])
        fused.append(k_mlp(ft, w1_ft, b1_eff[t:t + 1], _row(pf['ln_g']),
                           _row(pf['ln_b']), pf['w2'], _row(pf['b2'])))
    xs = jnp.stack(fused, 0)

    lp = params['lstm']
    bs = _row(lp['b_ih'] + lp['b_hh'])
    hw1 = jnp.stack([params['heads'][t]['w1'] for t in range(T_PRED)], 0)
    hb1 = jnp.stack([params['heads'][t]['b1'].reshape(1, -1)
                     for t in range(T_PRED)], 0)
    hw2 = jnp.stack([params['heads'][t]['w2'].reshape(1, -1)
                     for t in range(T_PRED)], 0)
    hb2 = jnp.stack([params['heads'][t]['b2'].reshape(1, 1)
                     for t in range(T_PRED)], 0)
    preds = k_lstm(xs, h0, lp['w_ih'].T, lp['w_hh'].T, bs,
                   hw1, hb1, hw2, hb2)
    return preds[:N_NODES]


# speedup vs baseline: 1.8823x; 1.1732x over previous
import functools

import jax
import jax.numpy as jnp
from jax import lax
from jax.experimental import pallas as pl
from jax.experimental.pallas import tpu as pltpu
from jax.experimental.pallas import tpu_sc as plsc

N_NODES = 10000
N_EDGES = 160000
D = 128
R = 5
T_PRED = 6
BLK = 512
NPAD = 10240
GRID = NPAD // BLK
TROWS = R * NPAD
CHUNK = 64
NCHUNK = 2
NS = 16
NC = 2
EPT = N_EDGES // NS
AB = 125
WNB = 40
NW = EPT // (WNB * AB)
CPT = N_EDGES // (NS * NC)
CNB = CPT // AB
SROWS = TROWS // NS
ZR = 100

_PREC = lax.Precision.HIGHEST


def _dot(a, b):
    return jnp.dot(a, b, preferred_element_type=jnp.float32, precision=_PREC)


def _pre_body(x_ref, w_ref, b_ref, m_ref, s_ref, root_ref, tbl_ref):
    x = (x_ref[:] - m_ref[:]) * s_ref[:]
    y = _dot(x, w_ref[:])
    root_ref[:] = y[:, :D] + b_ref[:]
    for c in range(NCHUNK):
        for r in range(R):
            lo = D + r * D + c * CHUNK
            tbl_ref[c, r] = y[:, lo:lo + CHUNK].astype(jnp.bfloat16)


@functools.partial(jax.jit, static_argnums=())
def k_pre(x, wcat, bias, m, s):
    return pl.pallas_call(
        _pre_body,
        grid=(GRID,),
        in_specs=[
            pl.BlockSpec((BLK, D), lambda i: (i, 0)),
            pl.BlockSpec((D, D * (R + 1)), lambda i: (0, 0)),
            pl.BlockSpec((1, D), lambda i: (0, 0)),
            pl.BlockSpec((1, D), lambda i: (0, 0)),
            pl.BlockSpec((1, D), lambda i: (0, 0)),
        ],
        out_specs=[
            pl.BlockSpec((BLK, D), lambda i: (i, 0)),
            pl.BlockSpec((NCHUNK, R, BLK, CHUNK), lambda i: (0, 0, i, 0)),
        ],
        out_shape=[
            jax.ShapeDtypeStruct((NPAD, D), jnp.float32),
            jax.ShapeDtypeStruct((NCHUNK, R, NPAD, CHUNK), jnp.bfloat16),
        ],
    )(x, wcat, bias, m, s)


def _comb_body(root_ref, acc_ref, cnt_ref, hpre_ref, stats_ref):
    i = pl.program_id(0)
    inv = 1.0 / jnp.maximum(cnt_ref[:], 1.0)
    h = root_ref[:]
    for r in range(R):
        m = jnp.concatenate([acc_ref[c, r] for c in range(NCHUNK)],
                            axis=1).astype(jnp.float32)
        h = h + m * inv[r][:, None]
    rows = lax.broadcasted_iota(jnp.int32, (BLK, 1), 0) + i * BLK
    h = jnp.where(rows < N_NODES, h, 0.0)
    hpre_ref[:] = h

    @pl.when(i == 0)
    def _():
        stats_ref[:] = jnp.zeros_like(stats_ref)

    stats_ref[0:1] += jnp.sum(h, axis=0, keepdims=True)
    stats_ref[1:2] += jnp.sum(h * h, axis=0, keepdims=True)


@jax.jit
def k_comb(root, acc, cnt):
    return pl.pallas_call(
        _comb_body,
        grid=(GRID,),
        in_specs=[
            pl.BlockSpec((BLK, D), lambda i: (i, 0)),
            pl.BlockSpec((NCHUNK, R, BLK, CHUNK), lambda i: (0, 0, i, 0)),
            pl.BlockSpec((R, BLK), lambda i: (0, i)),
        ],
        out_specs=[
            pl.BlockSpec((BLK, D), lambda i: (i, 0)),
            pl.BlockSpec((2, D), lambda i: (0, 0)),
        ],
        out_shape=[
            jax.ShapeDtypeStruct((NPAD, D), jnp.float32),
            jax.ShapeDtypeStruct((2, D), jnp.float32),
        ],
    )(root, acc, cnt)


def _bn_body(h_ref, mu_ref, rs_ref, g_ref, b_ref, a_ref, res_ref, o_ref):
    hb = (h_ref[:] - mu_ref[:]) * rs_ref[:] * g_ref[:] + b_ref[:]
    o = jnp.where(hb >= 0, hb, a_ref[0, 0] * hb)
    o_ref[:] = o + res_ref[:]


def _bn0_body(h_ref, mu_ref, rs_ref, g_ref, b_ref, a_ref, o_ref):
    hb = (h_ref[:] - mu_ref[:]) * rs_ref[:] * g_ref[:] + b_ref[:]
    o_ref[:] = jnp.where(hb >= 0, hb, a_ref[0, 0] * hb)


def _bn_specs(nin):
    vec = pl.BlockSpec((1, D), lambda i: (0, 0))
    blk = pl.BlockSpec((BLK, D), lambda i: (i, 0))
    scal = pl.BlockSpec((1, 1), lambda i: (0, 0))
    specs = [blk, vec, vec, vec, vec, scal] + [blk] * (nin - 6)
    return specs


@jax.jit
def k_bn(h, mu, rs, g, b, a, res):
    return pl.pallas_call(
        _bn_body,
        grid=(GRID,),
        in_specs=_bn_specs(7),
        out_specs=pl.BlockSpec((BLK, D), lambda i: (i, 0)),
        out_shape=jax.ShapeDtypeStruct((NPAD, D), jnp.float32),
    )(h, mu, rs, g, b, a, res)


@jax.jit
def k_bn0(h, mu, rs, g, b, a):
    return pl.pallas_call(
        _bn0_body,
        grid=(GRID,),
        in_specs=_bn_specs(6),
        out_specs=pl.BlockSpec((BLK, D), lambda i: (i, 0)),
        out_shape=jax.ShapeDtypeStruct((NPAD, D), jnp.float32),
    )(h, mu, rs, g, b, a)


def _mm_body(x_ref, w_ref, b_ref, o_ref):
    o_ref[:] = _dot(x_ref[:], w_ref[:]) + b_ref[:]


@jax.jit
def k_mm(x, w, b):
    ko, no = w.shape
    return pl.pallas_call(
        _mm_body,
        grid=(GRID,),
        in_specs=[
            pl.BlockSpec((BLK, ko), lambda i: (i, 0)),
            pl.BlockSpec((ko, no), lambda i: (0, 0)),
            pl.BlockSpec((1, no), lambda i: (0, 0)),
        ],
        out_specs=pl.BlockSpec((BLK, no), lambda i: (i, 0)),
        out_shape=jax.ShapeDtypeStruct((NPAD, no), jnp.float32),
    )(x, w, b)


def _mlp_body(x_ref, w1_ref, b1_ref, g_ref, bb_ref, w2_ref, b2_ref, o_ref):
    h = jnp.maximum(_dot(x_ref[:], w1_ref[:]) + b1_ref[:], 0.0)
    mu = jnp.mean(h, axis=1, keepdims=True)
    var = jnp.mean((h - mu) ** 2, axis=1, keepdims=True)
    hn = g_ref[:] * (h - mu) / jnp.sqrt(var + 1e-5) + bb_ref[:]
    o_ref[:] = _dot(hn, w2_ref[:]) + b2_ref[:]


@jax.jit
def k_mlp(x, w1, b1, g, bb, w2, b2):
    ki, kh = w1.shape
    ko = w2.shape[1]
    return pl.pallas_call(
        _mlp_body,
        grid=(GRID,),
        in_specs=[
            pl.BlockSpec((BLK, ki), lambda i: (i, 0)),
            pl.BlockSpec((ki, kh), lambda i: (0, 0)),
            pl.BlockSpec((1, kh), lambda i: (0, 0)),
            pl.BlockSpec((1, kh), lambda i: (0, 0)),
            pl.BlockSpec((1, kh), lambda i: (0, 0)),
            pl.BlockSpec((kh, ko), lambda i: (0, 0)),
            pl.BlockSpec((1, ko), lambda i: (0, 0)),
        ],
        out_specs=pl.BlockSpec((BLK, ko), lambda i: (i, 0)),
        out_shape=jax.ShapeDtypeStruct((NPAD, ko), jnp.float32),
    )(x, w1, b1, g, bb, w2, b2)


def _lstm_body(xs_ref, h0_ref, wih_ref, whh_ref, bs_ref,
               hw1_ref, hb1_ref, hw2_ref, hb2_ref, o_ref):
    h = h0_ref[:]
    c = jnp.zeros((BLK, D), jnp.float32)
    ps = []
    for t in range(T_PRED):
        gates = _dot(xs_ref[t], wih_ref[:]) + _dot(h, whh_ref[:]) + bs_ref[:]
        ii = gates[:, 0:D]
        ff = gates[:, D:2 * D]
        gg = gates[:, 2 * D:3 * D]
        oo = gates[:, 3 * D:4 * D]
        c = jax.nn.sigmoid(ff) * c + jax.nn.sigmoid(ii) * jnp.tanh(gg)
        h = jax.nn.sigmoid(oo) * jnp.tanh(c)
        hd = jnp.maximum(_dot(h, hw1_ref[t]) + hb1_ref[t], 0.0)
        p = jnp.sum(hd * hw2_ref[t], axis=1, keepdims=True) + hb2_ref[t]
        ps.append(p)
    o_ref[:] = jnp.concatenate(ps, axis=1)


@jax.jit
def k_lstm(xs, h0, wih, whh, bs, hw1, hb1, hw2, hb2):
    hh = 64
    return pl.pallas_call(
        _lstm_body,
        grid=(GRID,),
        in_specs=[
            pl.BlockSpec((T_PRED, BLK, D), lambda i: (0, i, 0)),
            pl.BlockSpec((BLK, D), lambda i: (i, 0)),
            pl.BlockSpec((D, 4 * D), lambda i: (0, 0)),
            pl.BlockSpec((D, 4 * D), lambda i: (0, 0)),
            pl.BlockSpec((1, 4 * D), lambda i: (0, 0)),
            pl.BlockSpec((T_PRED, D, hh), lambda i: (0, 0, 0)),
            pl.BlockSpec((T_PRED, 1, hh), lambda i: (0, 0, 0)),
            pl.BlockSpec((T_PRED, 1, hh), lambda i: (0, 0, 0)),
            pl.BlockSpec((T_PRED, 1, 1), lambda i: (0, 0, 0)),
        ],
        out_specs=pl.BlockSpec((BLK, T_PRED), lambda i: (i, 0)),
        out_shape=jax.ShapeDtypeStruct((NPAD, T_PRED), jnp.float32),
    )(xs, h0, wih, whh, bs, hw1, hb1, hw2, hb2)


def _agg_body(tbl_ref, g3_ref, h3_ref, out_ref, gi, hi, rows0, rows1, zb,
              acc, sem0, sem1):
    c = lax.axis_index("c")
    s = lax.axis_index("s")

    def zero_body(i, carry):
        zb[i, pl.ds(0, 32)] = jnp.zeros((32,), jnp.bfloat16)
        zb[i, pl.ds(32, 32)] = jnp.zeros((32,), jnp.bfloat16)
        return carry

    lax.fori_loop(0, ZR, zero_body, 0)
    for p in range(NCHUNK // NC):
        q = p * NC + c

        def zfire(j, carry):
            pltpu.async_copy(zb, acc.at[pl.ds(s * SROWS + j * ZR, ZR)], sem0)
            return carry

        def zdrain(j, carry):
            pltpu.make_async_copy(zb, acc.at[pl.ds(s * SROWS, ZR)],
                                  sem0).wait()
            return carry

        lax.fori_loop(0, SROWS // ZR, zfire, 0)
        lax.fori_loop(0, SROWS // ZR, zdrain, 0)
        plsc.subcore_barrier()
        tq = tbl_ref.at[q]
        for w in range(NW):
            pltpu.sync_copy(g3_ref.at[s, w], gi)
            pltpu.sync_copy(h3_ref.at[s, w], hi)
            pltpu.async_copy(tq.at[gi.at[0]], rows0, sem0)

            def body2(jj, carry):
                j0 = jj * 2
                pltpu.async_copy(tq.at[gi.at[j0 + 1]], rows1, sem1)
                pltpu.make_async_copy(tq.at[gi.at[j0]], rows0, sem0).wait()
                pltpu.sync_copy(rows0, acc.at[hi.at[j0]], add=True)
                j2 = jnp.minimum(j0 + 2, WNB - 1)
                pltpu.async_copy(tq.at[gi.at[j2]], rows0, sem0)
                pltpu.make_async_copy(tq.at[gi.at[j0 + 1]], rows1,
                                      sem1).wait()
                pltpu.sync_copy(rows1, acc.at[hi.at[j0 + 1]], add=True)
                return carry

            lax.fori_loop(0, WNB // 2, body2, 0)
            pltpu.make_async_copy(tq.at[gi.at[WNB - 1]], rows0, sem0).wait()
        plsc.subcore_barrier()
        pltpu.sync_copy(acc.at[pl.ds(s * SROWS, SROWS)],
                        out_ref.at[q].at[pl.ds(s * SROWS, SROWS)])
        plsc.subcore_barrier()


@functools.lru_cache(maxsize=None)
def _sc_agg_kernel():
    mesh = plsc.VectorSubcoreMesh(core_axis_name="c", subcore_axis_name="s")
    return pl.kernel(
        _agg_body,
        out_type=jax.ShapeDtypeStruct((NCHUNK, TROWS, CHUNK), jnp.bfloat16),
        mesh=mesh,
        compiler_params=pltpu.CompilerParams(use_tc_tiling_on_sc=False),
        scratch_types=[
            pltpu.VMEM((WNB, AB), jnp.int32),
            pltpu.VMEM((WNB, AB), jnp.int32),
            pltpu.VMEM((AB, CHUNK), jnp.bfloat16),
            pltpu.VMEM((AB, CHUNK), jnp.bfloat16),
            pltpu.VMEM((ZR, CHUNK), jnp.bfloat16),
            pltpu.VMEM_SHARED((TROWS, CHUNK), jnp.bfloat16),
            pltpu.SemaphoreType.DMA,
            pltpu.SemaphoreType.DMA,
        ],
    )


def sc_agg(tbl3, gidx, h3):
    return _sc_agg_kernel()(tbl3, gidx, h3)


def _cnt_body(h32_ref, ones_ref, out_ref, hi, ones_v, zb, cacc, sem):
    c = lax.axis_index("c")
    s = lax.axis_index("s")
    w = c * NS + s
    pltpu.sync_copy(h32_ref.at[w], hi)
    pltpu.sync_copy(ones_ref, ones_v)

    def zero_body(i, carry):
        zb[i, pl.ds(0, 16)] = jnp.zeros((16,), jnp.float32)
        return carry

    lax.fori_loop(0, ZR, zero_body, 0)

    def zcp(j, carry):
        pltpu.sync_copy(zb, cacc.at[pl.ds(s * SROWS + j * ZR, ZR)])
        return carry

    lax.fori_loop(0, SROWS // ZR, zcp, 0)
    plsc.subcore_barrier()

    def body(j, carry):
        pltpu.sync_copy(ones_v.at[pl.ds(0, AB)], cacc.at[hi.at[j]], add=True)
        return carry

    lax.fori_loop(0, CNB, body, 0)
    plsc.subcore_barrier()
    pltpu.sync_copy(cacc.at[pl.ds(s * SROWS, SROWS)],
                    out_ref.at[c].at[pl.ds(s * SROWS, SROWS)])


@functools.lru_cache(maxsize=None)
def _sc_cnt_kernel():
    mesh = plsc.VectorSubcoreMesh(core_axis_name="c", subcore_axis_name="s")
    return pl.kernel(
        _cnt_body,
        out_type=jax.ShapeDtypeStruct((NC, TROWS, 16), jnp.float32),
        mesh=mesh,
        compiler_params=pltpu.CompilerParams(use_tc_tiling_on_sc=False),
        scratch_types=[
            pltpu.VMEM((CNB, AB), jnp.int32),
            pltpu.VMEM((128, 16), jnp.float32),
            pltpu.VMEM((ZR, 16), jnp.float32),
            pltpu.VMEM_SHARED((TROWS, 16), jnp.float32),
            pltpu.SemaphoreType.DMA,
        ],
    )


def sc_cnt(h32, ones):
    return _sc_cnt_kernel()(h32, ones)


def _layer_norm(h, g, b):
    mu = h.mean(-1, keepdims=True)
    var = ((h - mu) ** 2).mean(-1, keepdims=True)
    return g * (h - mu) / jnp.sqrt(var + 1e-5) + b


def _mlp_small(x, p):
    h = jnp.maximum(x @ p['w1'] + p['b1'], 0.0)
    h = _layer_norm(h, p['ln_g'], p['ln_b'])
    return h @ p['w2'] + p['b2']


def _row(v):
    return v.reshape(1, -1)


def kernel(x_seq, edge_index, edge_attr, graph_global_env_features,
           timeline_time_features, params):
    f32 = jnp.float32
    src = edge_index[0].astype(jnp.int32)
    dst = edge_index[1].astype(jnp.int32)
    etype = edge_attr[:, 4].astype(jnp.int32)
    gidx = (etype * NPAD + src).reshape(NS, NW, WNB, AB)
    hflat = etype * NPAD + dst
    h3 = hflat.reshape(NS, NW, WNB, AB)
    h32 = hflat.reshape(NS * NC, CNB, AB)

    cnt2 = sc_cnt(h32, jnp.ones((128, 16), f32))
    cnt = (cnt2[0, :, 0] + cnt2[1, :, 0]).reshape(R, NPAD)

    xpad = jnp.pad(x_seq, ((0, 0), (0, NPAD - N_NODES), (0, 0)))
    mean = _row(params['feat_mean'])
    std = _row(params['feat_std'] + 1e-8)
    inv_std = 1.0 / std
    zero_m = jnp.zeros((1, D), f32)
    one_s = jnp.ones((1, D), f32)

    def wcat_of(bp):
        return jnp.concatenate([bp['w_root']] +
                               [bp['w_rel'][r] for r in range(R)], axis=1)

    def conv(x, bp, m, s):
        root, tbl = k_pre(x, wcat_of(bp), _row(bp['bias']), m, s)
        acc = sc_agg(tbl.reshape(NCHUNK, TROWS, CHUNK), gidx, h3)
        h_pre, stats = k_comb(root, acc.reshape(NCHUNK, R, NPAD, CHUNK), cnt)
        mu = stats[0] / N_NODES
        var = stats[1] / N_NODES - mu * mu
        rstd = 1.0 / jnp.sqrt(var + 1e-5)
        return h_pre, _row(mu), _row(rstd)

    def gen(x, gp):
        b0 = gp['block0']
        hp, mu, rstd = conv(x, b0, mean, inv_std)
        a0 = b0['prelu'].reshape(1, 1)
        h0 = k_bn0(hp, mu, rstd, _row(b0['bn_g']), _row(b0['bn_b']), a0)
        b1 = gp['block1']
        hp1, mu1, rstd1 = conv(h0, b1, zero_m, one_s)
        a1 = b1['prelu'].reshape(1, 1)
        h1 = k_bn(hp1, mu1, rstd1, _row(b1['bn_g']), _row(b1['bn_b']), a1, h0)
        return k_mm(h1, gp['fc_w'], _row(gp['fc_b']))

    f0 = gen(xpad[0], params['gen_h0'])
    pe = params['h0_enc']
    h0 = k_mlp(f0, pe['w1'], _row(pe['b1']), _row(pe['ln_g']),
               _row(pe['ln_b']), pe['w2'], _row(pe['b2']))

    env_emb = _mlp_small(graph_global_env_features, params['env_enc'])
    tim_emb = _mlp_small(timeline_time_features, params['time_enc'])
    pf = params['fusion']
    w1_ft = pf['w1'][:D]
    b1_eff = (_row(pf['b1']) + env_emb @ pf['w1'][D:D + 16]
              + tim_emb @ pf['w1'][D + 16:])

    fused = []
    for t in range(T_PRED):
        ft = gen(xpad[t + 1], params['gen_seq'])
        fused.append(k_mlp(ft, w1_ft, b1_eff[t:t + 1], _row(pf['ln_g']),
                           _row(pf['ln_b']), pf['w2'], _row(pf['b2'])))
    xs = jnp.stack(fused, 0)

    lp = params['lstm']
    bs = _row(lp['b_ih'] + lp['b_hh'])
    hw1 = jnp.stack([params['heads'][t]['w1'] for t in range(T_PRED)], 0)
    hb1 = jnp.stack([params['heads'][t]['b1'].reshape(1, -1)
                     for t in range(T_PRED)], 0)
    hw2 = jnp.stack([params['heads'][t]['w2'].reshape(1, -1)
                     for t in range(T_PRED)], 0)
    hb2 = jnp.stack([params['heads'][t]['b2'].reshape(1, 1)
                     for t in range(T_PRED)], 0)
    preds = k_lstm(xs, h0, lp['w_ih'].T, lp['w_hh'].T, bs,
                   hw1, hb1, hw2, hb2)
    return preds[:N_NODES]
